# baseline (speedup 1.0000x reference)
import functools

import jax
import jax.numpy as jnp
from jax import lax
from jax.experimental import pallas as pl
from jax.experimental.pallas import tpu as pltpu
from jax.experimental.pallas import tpu_sc as plsc

_TN = 4096
_IMAX = 2**31 - 1


def _nn_body(nrows, mius_ref, tab_ref, soft_ref, keys_ref, tl_ref, rarg_ref):
    pid = pl.program_id(0)
    nprog = pl.num_programs(0)
    bsz, tn = mius_ref.shape[0], tab_ref.shape[0]

    @pl.when(pid == 0)
    def _init():
        rarg_ref[...] = jnp.full((bsz, 1), _IMAX, jnp.int32)

    q = mius_ref[...]
    t = tab_ref[...]
    q2h = 0.5 * jnp.sum(q * q, axis=1, keepdims=True)
    t2h = 0.5 * jnp.sum(t * t, axis=1, keepdims=True).T
    limit = nrows - pid * tn
    col1 = lax.broadcasted_iota(jnp.int32, (1, tn), 1)
    t2h = t2h + jnp.where(col1 < limit, 0.0, jnp.inf)
    dot = lax.dot_general(
        q.astype(jnp.bfloat16), t.astype(jnp.bfloat16),
        dimension_numbers=(((1,), (1,)), ((), ())),
        preferred_element_type=jnp.float32)
    hit = dot >= (q2h + t2h)
    colf = col1.astype(jnp.float32)
    locf = jnp.min(jnp.where(hit, colf, jnp.inf), axis=1, keepdims=True)
    loc = jnp.where(locf == jnp.inf, _IMAX,
                    locf.astype(jnp.int32) + pid * tn)
    rarg_ref[...] = jnp.minimum(rarg_ref[...], loc)

    @pl.when(pid == nprog - 1)
    def _finish():
        keys_ref[...] = jnp.where(rarg_ref[...] == _IMAX, -1, rarg_ref[...])
        s = soft_ref[...]
        rmax = jnp.max(s, axis=1, keepdims=True)
        cc = lax.broadcasted_iota(jnp.int32, s.shape, 1)
        tl_ref[...] = jnp.min(jnp.where(s == rmax, cc, _IMAX), axis=1,
                              keepdims=True)


def _nn_keys(mius, miu_table, softlabels):
    B, LAT = mius.shape
    N = miu_table.shape[0]
    C = softlabels.shape[1]
    ntiles = (N + _TN - 1) // _TN
    keys, tl = pl.pallas_call(
        functools.partial(_nn_body, N),
        grid=(ntiles,),
        in_specs=[
            pl.BlockSpec((B, LAT), lambda i: (0, 0)),
            pl.BlockSpec((_TN, LAT), lambda i: (i, 0)),
            pl.BlockSpec((B, C), lambda i: (0, 0)),
        ],
        out_specs=[
            pl.BlockSpec((B, 1), lambda i: (0, 0)),
            pl.BlockSpec((B, 1), lambda i: (0, 0)),
        ],
        out_shape=[
            jax.ShapeDtypeStruct((B, 1), jnp.int32),
            jax.ShapeDtypeStruct((B, 1), jnp.int32),
        ],
        scratch_shapes=[pltpu.VMEM((B, 1), jnp.int32)],
        compiler_params=pltpu.CompilerParams(
            dimension_semantics=("arbitrary",)),
    )(mius, miu_table, softlabels)
    return keys.reshape(B), tl.reshape(B)


def _sc_swap(B, C, NCLS):
    info = plsc.get_sparse_core_info()
    NC, NS, L = info.num_cores, info.num_subcores, info.num_lanes
    NW = NC * NS
    bpw = B // NW
    mesh = plsc.VectorSubcoreMesh(core_axis_name="c", subcore_axis_name="s")

    @functools.partial(
        pl.kernel, mesh=mesh,
        out_type=jax.ShapeDtypeStruct((B * C,), jnp.float32),
        scratch_types=[
            pltpu.VMEM((bpw,), jnp.int32),
            pltpu.VMEM((bpw,), jnp.int32),
            pltpu.VMEM((bpw,), jnp.int32),
            pltpu.VMEM((bpw,), jnp.int32),
            pltpu.VMEM((bpw,), jnp.int32),
            pltpu.VMEM((bpw,), jnp.int32),
            pltpu.VMEM((bpw * C,), jnp.float32),
            pltpu.SemaphoreType.DMA,
            pltpu.SemaphoreType.DMA,
        ],
        compiler_params=pltpu.CompilerParams(needs_layout_passes=False))
    def k(keys_hbm, tl_hbm, ftab_hbm, foff_hbm, new_hbm, out_hbm,
          keys_v, idx_v, ftk_v, tl_v, ftb_v, off_v, rows_v, sem, sem2):
        wid = lax.axis_index("s") * NC + lax.axis_index("c")
        base = wid * bpw
        ck = pltpu.async_copy(keys_hbm.at[pl.ds(base, bpw)], keys_v, sem)
        c1 = pltpu.async_copy(tl_hbm.at[pl.ds(base, bpw)], tl_v, sem2)
        c2 = pltpu.async_copy(ftab_hbm.at[pl.ds(base, bpw)], ftb_v, sem2)
        c3 = pltpu.async_copy(foff_hbm.at[pl.ds(base, bpw)], off_v, sem2)
        c4 = pltpu.async_copy(new_hbm.at[pl.ds(base * C, bpw * C)], rows_v,
                              sem2)
        ck.wait()
        for g in range(bpw // L):
            sl = pl.ds(g * L, L)
            idx_v[sl] = jnp.maximum(keys_v[sl], 0)
        gat = pltpu.async_copy(ftab_hbm.at[idx_v], ftk_v, sem)
        c1.wait(); c2.wait(); c3.wait(); c4.wait()
        gat.wait()
        for g in range(bpw // L):
            sl = pl.ds(g * L, L)
            kv = keys_v[sl]
            tl = tl_v[sl]
            valid = kv != -1
            member = valid & (ftk_v[sl] == 1)
            offs = jnp.where(valid & (ftb_v[sl] == 1), off_v[sl], 0)
            fake = lax.rem(tl + offs, NCLS)
            rbase = (lax.iota(jnp.int32, L) + g * L) * C
            vt = plsc.load_gather(rows_v, [rbase + tl])
            vf = plsc.load_gather(rows_v, [rbase + fake])
            plsc.store_scatter(rows_v, [rbase + tl], vf, mask=member)
            plsc.store_scatter(rows_v, [rbase + fake], vt, mask=member)
        pltpu.sync_copy(rows_v, out_hbm.at[pl.ds(base * C, bpw * C)])

    return k


def kernel(mius, logvars, softlabels, new_softlabels, miu_table, flip_table,
           flip_offset):
    B = mius.shape[0]
    C = new_softlabels.shape[1]
    keys, true_labels = _nn_keys(mius, miu_table, softlabels)
    swap = _sc_swap(B, C, C)
    out = swap(keys, true_labels, flip_table, flip_offset,
               new_softlabels.reshape(B * C))
    return out.reshape(B, C)


# speedup vs baseline: 1.6074x; 1.6074x over previous
import functools

import jax
import jax.numpy as jnp
from jax import lax
from jax.experimental import pallas as pl
from jax.experimental.pallas import tpu as pltpu
from jax.experimental.pallas import tpu_sc as plsc

_TN = 8192
_IMAX = 2**31 - 1


def _nn_body(nrows, mius_ref, tab_ref, soft_ref, keys_ref, tl_ref, rarg_ref):
    pid = pl.program_id(0)
    nprog = pl.num_programs(0)
    bsz, tn = mius_ref.shape[0], tab_ref.shape[1]

    @pl.when(pid == 0)
    def _init():
        rarg_ref[...] = jnp.full((bsz, 1), _IMAX, jnp.int32)

    q = mius_ref[...]
    t = tab_ref[...]
    q2h = 0.5 * jnp.sum(q * q, axis=1, keepdims=True)
    t2h = 0.5 * jnp.sum(t * t, axis=0, keepdims=True)
    limit = nrows - pid * tn
    col1 = lax.broadcasted_iota(jnp.int32, (1, tn), 1)
    t2h = t2h + jnp.where(col1 < limit, 0.0, jnp.inf)
    dot = lax.dot_general(
        q.astype(jnp.bfloat16), t.astype(jnp.bfloat16),
        dimension_numbers=(((1,), (0,)), ((), ())),
        preferred_element_type=jnp.float32)
    hit = dot >= (q2h + t2h)
    colf = col1.astype(jnp.float32)
    locf = jnp.min(jnp.where(hit, colf, jnp.inf), axis=1, keepdims=True)
    loc = jnp.where(locf == jnp.inf, _IMAX,
                    locf.astype(jnp.int32) + pid * tn)
    rarg_ref[...] = jnp.minimum(rarg_ref[...], loc)

    @pl.when(pid == nprog - 1)
    def _finish():
        keys_ref[...] = jnp.where(rarg_ref[...] == _IMAX, -1, rarg_ref[...])
        s = soft_ref[...]
        rmax = jnp.max(s, axis=1, keepdims=True)
        cc = lax.broadcasted_iota(jnp.int32, s.shape, 1)
        tl_ref[...] = jnp.min(jnp.where(s == rmax, cc, _IMAX), axis=1,
                              keepdims=True)


def _nn_keys(mius, miu_table, softlabels):
    B, LAT = mius.shape
    N = miu_table.shape[0]
    C = softlabels.shape[1]
    ntiles = (N + _TN - 1) // _TN
    keys, tl = pl.pallas_call(
        functools.partial(_nn_body, N),
        grid=(ntiles,),
        in_specs=[
            pl.BlockSpec((B, LAT), lambda i: (0, 0)),
            pl.BlockSpec((LAT, _TN), lambda i: (0, i)),
            pl.BlockSpec((B, C), lambda i: (0, 0)),
        ],
        out_specs=[
            pl.BlockSpec((B, 1), lambda i: (0, 0)),
            pl.BlockSpec((B, 1), lambda i: (0, 0)),
        ],
        out_shape=[
            jax.ShapeDtypeStruct((B, 1), jnp.int32),
            jax.ShapeDtypeStruct((B, 1), jnp.int32),
        ],
        scratch_shapes=[pltpu.VMEM((B, 1), jnp.int32)],
        compiler_params=pltpu.CompilerParams(
            dimension_semantics=("arbitrary",)),
    )(mius, miu_table.T, softlabels)
    return keys.reshape(B), tl.reshape(B)


def _sc_swap(B, C, NCLS):
    info = plsc.get_sparse_core_info()
    NC, NS, L = info.num_cores, info.num_subcores, info.num_lanes
    NW = NC * NS
    bpw = B // NW
    mesh = plsc.VectorSubcoreMesh(core_axis_name="c", subcore_axis_name="s")

    @functools.partial(
        pl.kernel, mesh=mesh,
        out_type=jax.ShapeDtypeStruct((B * C,), jnp.float32),
        scratch_types=[
            pltpu.VMEM((bpw,), jnp.int32),
            pltpu.VMEM((bpw,), jnp.int32),
            pltpu.VMEM((bpw,), jnp.int32),
            pltpu.VMEM((bpw,), jnp.int32),
            pltpu.VMEM((bpw,), jnp.int32),
            pltpu.VMEM((bpw,), jnp.int32),
            pltpu.VMEM((bpw * C,), jnp.float32),
            pltpu.SemaphoreType.DMA,
            pltpu.SemaphoreType.DMA,
        ],
        compiler_params=pltpu.CompilerParams(needs_layout_passes=False))
    def k(keys_hbm, tl_hbm, ftab_hbm, foff_hbm, new_hbm, out_hbm,
          keys_v, idx_v, ftk_v, tl_v, ftb_v, off_v, rows_v, sem, sem2):
        wid = lax.axis_index("s") * NC + lax.axis_index("c")
        base = wid * bpw
        ck = pltpu.async_copy(keys_hbm.at[pl.ds(base, bpw)], keys_v, sem)
        c1 = pltpu.async_copy(tl_hbm.at[pl.ds(base, bpw)], tl_v, sem2)
        c2 = pltpu.async_copy(ftab_hbm.at[pl.ds(base, bpw)], ftb_v, sem2)
        c3 = pltpu.async_copy(foff_hbm.at[pl.ds(base, bpw)], off_v, sem2)
        c4 = pltpu.async_copy(new_hbm.at[pl.ds(base * C, bpw * C)], rows_v,
                              sem2)
        ck.wait()
        for g in range(bpw // L):
            sl = pl.ds(g * L, L)
            idx_v[sl] = jnp.maximum(keys_v[sl], 0)
        gat = pltpu.async_copy(ftab_hbm.at[idx_v], ftk_v, sem)
        c1.wait(); c2.wait(); c3.wait(); c4.wait()
        gat.wait()
        for g in range(bpw // L):
            sl = pl.ds(g * L, L)
            kv = keys_v[sl]
            tl = tl_v[sl]
            valid = kv != -1
            member = valid & (ftk_v[sl] == 1)
            offs = jnp.where(valid & (ftb_v[sl] == 1), off_v[sl], 0)
            fake = lax.rem(tl + offs, NCLS)
            rbase = (lax.iota(jnp.int32, L) + g * L) * C
            vt = plsc.load_gather(rows_v, [rbase + tl])
            vf = plsc.load_gather(rows_v, [rbase + fake])
            plsc.store_scatter(rows_v, [rbase + tl], vf, mask=member)
            plsc.store_scatter(rows_v, [rbase + fake], vt, mask=member)
        pltpu.sync_copy(rows_v, out_hbm.at[pl.ds(base * C, bpw * C)])

    return k


def kernel(mius, logvars, softlabels, new_softlabels, miu_table, flip_table,
           flip_offset):
    B = mius.shape[0]
    C = new_softlabels.shape[1]
    keys, true_labels = _nn_keys(mius, miu_table, softlabels)
    swap = _sc_swap(B, C, C)
    out = swap(keys, true_labels, flip_table, flip_offset,
               new_softlabels.reshape(B * C))
    return out.reshape(B, C)
